# baseline (speedup 1.0000x reference)
import functools

import jax
import jax.numpy as jnp
from jax import lax
from jax.experimental import pallas as pl
from jax.experimental.pallas import tpu as pltpu
from jax.experimental.pallas import tpu_sc as plsc

_C, _H, _W = 5, 512, 512
_NT = 1024
_NW = 32
_RW = _NT // _NW
_BAND = 8
_NB = _RW // _BAND
_SRC = 9
_PAD = 16
_WLEN = _SRC * _NT
_BLEN = _BAND * _NT


def _img_body(sh_ref, img_ref, out_ref):
    v = sh_ref[0]
    h = sh_ref[1]
    x = img_ref[...]
    x = pltpu.roll(x, jnp.mod(v, _H), 1)
    x = pltpu.roll(x, jnp.mod(h, _W), 2)
    out_ref[...] = x


def _sc_cat(sh_hbm, n_hbm, l0_hbm, l1_hbm, f_hbm,
            no_hbm, lo0_hbm, lo1_hbm, fo_hbm,
            shv, n_in, l0_in, l1_in, f_in, cnt_b, lo0_b, lo1_b, fo_b):
    wid = lax.axis_index("s") * 2 + lax.axis_index("c")
    pltpu.sync_copy(sh_hbm, shv)
    shvec = shv[...]
    vs = shvec[0]
    hs = shvec[1]
    dv = 2 * vs
    dh = 2 * hs
    vsf = lax.convert_element_type(vs, jnp.float32)
    hsf = lax.convert_element_type(hs, jnp.float32)
    dvf = vsf * 2.0
    dhf = hsf * 2.0
    iota = lax.broadcasted_iota(jnp.int32, (16,), 0)
    zf = jnp.zeros((16,), jnp.float32)

    def band_body(b, carry):
        bs = wid * _RW + b * _BAND
        ws = jnp.clip(bs - dv - 1, 0, _NT - _SRC)
        src_off = pl.multiple_of(ws * _NT, 128)
        dst = pl.ds(_PAD, _WLEN)
        pltpu.sync_copy(n_hbm.at[pl.ds(src_off, _WLEN)], n_in.at[dst])
        pltpu.sync_copy(l0_hbm.at[pl.ds(src_off, _WLEN)], l0_in.at[dst])
        pltpu.sync_copy(l1_hbm.at[pl.ds(src_off, _WLEN)], l1_in.at[dst])
        pltpu.sync_copy(f_hbm.at[pl.ds(src_off, _WLEN)], f_in.at[dst])

        def row_body(rd, c2):
            dest = bs + rd

            def chunk(c, c3):
                base = c * 16
                jd = base + iota
                acc_n = zf
                acc_f = zf
                acc_0 = zf
                acc_1 = zf
                for r0 in (0, 1):
                    s = dest - dv - r0
                    rp = s - ws
                    sf = lax.convert_element_type(s, jnp.float32)
                    srow_f = jnp.where(
                        (s >= 0) & (s < _NT) & (rp >= 0) & (rp < _SRC),
                        jnp.float32(1.0), jnp.float32(0.0))
                    thr0 = sf + dvf + 1.0
                    sub0 = sf + dvf + jnp.float32(r0)
                    for r1 in (0, 1):
                        off = rp * _NT + base - dh - r1 + _PAD
                        off = jnp.clip(off, 0, _PAD + _WLEN - 16)
                        jv = jd - dh - r1
                        nv = n_in[pl.ds(off, 16)]
                        fv = f_in[pl.ds(off, 16)]
                        l0v = l0_in[pl.ds(off, 16)]
                        l1v = l1_in[pl.ds(off, 16)]
                        jf = lax.convert_element_type(jv, jnp.float32)
                        u0 = ((sf + l0v) * 0.5 + vsf) * 2.0
                        u1 = ((jf + l1v) * 0.5 + hsf) * 2.0
                        b0f = jnp.where(u0 >= thr0, 1.0, 0.0)
                        b1f = jnp.where(u1 >= jf + dhf + 1.0, 1.0, 0.0)
                        m0f = b0f if r0 else 1.0 - b0f
                        m1f = b1f if r1 else 1.0 - b1f
                        onf = jnp.where((nv > 0) & (jv >= 0) & (jv < _NT),
                                        1.0, 0.0)
                        mf = onf * m0f * m1f * srow_f
                        nl0 = u0 - sub0
                        nl1 = u1 - (jf + dhf + jnp.float32(r1))
                        acc_n = acc_n + mf
                        acc_f = acc_f + fv * mf
                        acc_0 = acc_0 + nl0 * mf
                        acc_1 = acc_1 + nl1 * mf
                ob = pl.ds(rd * _NT + base, 16)
                cnt_b[ob] = jnp.minimum(acc_n, 1.0).astype(jnp.int32)
                fo_b[ob] = acc_f
                lo0_b[ob] = acc_0
                lo1_b[ob] = acc_1
                return c3
            return lax.fori_loop(0, 64, chunk, c2)
        lax.fori_loop(0, _BAND, row_body, 0)

        out_off = pl.multiple_of(bs * _NT, 128)
        pltpu.sync_copy(cnt_b, no_hbm.at[pl.ds(out_off, _BLEN)])
        pltpu.sync_copy(fo_b, fo_hbm.at[pl.ds(out_off, _BLEN)])
        pltpu.sync_copy(lo0_b, lo0_hbm.at[pl.ds(out_off, _BLEN)])
        pltpu.sync_copy(lo1_b, lo1_hbm.at[pl.ds(out_off, _BLEN)])
        return carry
    lax.fori_loop(0, _NB, band_body, 0)


def kernel(images, psf_params, n_sources, locs, fluxes, vertical_shift,
           horizontal_shift):
    v = jnp.asarray(vertical_shift, jnp.int32)
    h = jnp.asarray(horizontal_shift, jnp.int32)
    sh = jnp.stack([v, h])

    img = pl.pallas_call(
        _img_body,
        out_shape=jax.ShapeDtypeStruct((_C, _H, _W), jnp.float32),
        in_specs=[pl.BlockSpec(memory_space=pltpu.SMEM),
                  pl.BlockSpec((_C, _H, _W), lambda: (0, 0, 0))],
        out_specs=pl.BlockSpec((_C, _H, _W), lambda: (0, 0, 0)),
    )(sh, images)

    sh16 = jnp.zeros((16,), jnp.int32).at[0].set(v).at[1].set(h)
    mesh = plsc.VectorSubcoreMesh(core_axis_name="c", subcore_axis_name="s")
    fvec = jax.ShapeDtypeStruct((_NT * _NT,), jnp.float32)
    sc_cat = functools.partial(
        pl.kernel,
        mesh=mesh,
        out_type=(jax.ShapeDtypeStruct((_NT * _NT,), jnp.int32),
                  fvec, fvec, fvec),
        scratch_types=[
            pltpu.VMEM((16,), jnp.int32),
            pltpu.VMEM((_WLEN + 2 * _PAD,), jnp.int32),
            pltpu.VMEM((_WLEN + 2 * _PAD,), jnp.float32),
            pltpu.VMEM((_WLEN + 2 * _PAD,), jnp.float32),
            pltpu.VMEM((_WLEN + 2 * _PAD,), jnp.float32),
            pltpu.VMEM((_BLEN,), jnp.int32),
            pltpu.VMEM((_BLEN,), jnp.float32),
            pltpu.VMEM((_BLEN,), jnp.float32),
            pltpu.VMEM((_BLEN,), jnp.float32),
        ],
    )(_sc_cat)
    n_out, lo0, lo1, f_out = sc_cat(
        sh16, n_sources.reshape(-1), locs[:, :, 0, 0].reshape(-1),
        locs[:, :, 0, 1].reshape(-1), fluxes.reshape(-1))

    locs_out = jnp.stack([lo0.reshape(_NT, _NT), lo1.reshape(_NT, _NT)],
                         axis=-1).reshape(_NT, _NT, 1, 2)
    return (img, psf_params, locs_out,
            f_out.reshape(_NT, _NT, 1, 1), n_out.reshape(_NT, _NT))


# speedup vs baseline: 1.1171x; 1.1171x over previous
import functools

import jax
import jax.numpy as jnp
from jax import lax
from jax.experimental import pallas as pl
from jax.experimental.pallas import tpu as pltpu
from jax.experimental.pallas import tpu_sc as plsc

_C, _H, _W = 5, 512, 512
_NT = 1024
_NW = 32
_RW = _NT // _NW
_BAND = 8
_NB = _RW // _BAND
_SRC = 9
_PAD = 16
_WLEN = _SRC * _NT
_BLEN = _BAND * _NT


def _img_body(sh_ref, img_ref, out_ref):
    v = sh_ref[0]
    h = sh_ref[1]
    x = img_ref[...]
    x = pltpu.roll(x, jnp.mod(v, _H), 1)
    x = pltpu.roll(x, jnp.mod(h, _W), 2)
    out_ref[...] = x


def _sc_cat(sh_hbm, n_hbm, l0_hbm, l1_hbm, f_hbm,
            no_hbm, lo0_hbm, lo1_hbm, fo_hbm,
            shv, n_in, l0_in, l1_in, f_in, cnt_b, lo0_b, lo1_b, fo_b):
    wid = lax.axis_index("s") * 2 + lax.axis_index("c")
    pltpu.sync_copy(sh_hbm, shv)
    shvec = shv[...]
    vs = shvec[0]
    hs = shvec[1]
    dv = 2 * vs
    dh = 2 * hs
    vsf = lax.convert_element_type(vs, jnp.float32)
    hsf = lax.convert_element_type(hs, jnp.float32)
    dvf = vsf * 2.0
    dhf = hsf * 2.0
    iota = lax.broadcasted_iota(jnp.int32, (16,), 0)
    zf = jnp.zeros((16,), jnp.float32)

    def band_body(b, carry):
        bs = wid * _RW + b * _BAND
        ws = jnp.clip(bs - dv - 1, 0, _NT - _SRC)
        src_off = pl.multiple_of(ws * _NT, 128)
        dst = pl.ds(_PAD, _WLEN)
        pltpu.sync_copy(n_hbm.at[pl.ds(src_off, _WLEN)], n_in.at[dst])
        pltpu.sync_copy(l0_hbm.at[pl.ds(src_off, _WLEN)], l0_in.at[dst])
        pltpu.sync_copy(l1_hbm.at[pl.ds(src_off, _WLEN)], l1_in.at[dst])
        pltpu.sync_copy(f_hbm.at[pl.ds(src_off, _WLEN)], f_in.at[dst])

        def row_body(rd, c2):
            dest = bs + rd
            s0 = dest - dv
            s1 = dest - dv - 1
            rp0 = s0 - ws
            rp1 = s1 - ws
            sf0 = lax.convert_element_type(s0, jnp.float32)
            sf1 = lax.convert_element_type(s1, jnp.float32)

            off0r = jnp.clip(rp0 * _NT + _PAD, 0, _PAD + _WLEN - _NT)
            off1r = jnp.clip(rp1 * _NT + _PAD, 0, _PAD + _WLEN - _NT)
            thr0c = sf1 + dvf + 1.0

            def checkc(c, acc):
                cb = c * 16
                l0v = l0_in[pl.ds(off1r + cb, 16)]
                l1v = l1_in[pl.ds(off0r + cb, 16)]
                jfc = lax.convert_element_type(iota + cb, jnp.float32)
                u0 = ((sf1 + l0v) * 0.5 + vsf) * 2.0
                u1 = ((jfc + l1v) * 0.5 + hsf) * 2.0
                b0f = jnp.where(u0 >= thr0c, 1.0, 0.0)
                b1f = jnp.where(u1 >= jfc + dhf + 1.0, 1.0, 0.0)
                return jnp.maximum(acc, jnp.maximum(b0f, b1f))
            accfl = lax.fori_loop(0, 64, checkc, zf)
            accs = accfl[0]
            for _k in range(1, 16):
                accs = accs + accfl[_k]
            anyflag = accs > 0.0

            def base_chunk(c, c3):
                base = c * 16
                jd = base + iota
                rp = rp0
                sf = sf0
                srow_f = jnp.where(
                    (s0 >= 0) & (s0 < _NT) & (rp >= 0) & (rp < _SRC),
                    jnp.float32(1.0), jnp.float32(0.0))
                off = jnp.clip(rp * _NT + base - dh + _PAD,
                               0, _PAD + _WLEN - 16)
                jv = jd - dh
                nv = n_in[pl.ds(off, 16)]
                fv = f_in[pl.ds(off, 16)]
                l0v = l0_in[pl.ds(off, 16)]
                l1v = l1_in[pl.ds(off, 16)]
                jf = lax.convert_element_type(jv, jnp.float32)
                u0 = ((sf + l0v) * 0.5 + vsf) * 2.0
                u1 = ((jf + l1v) * 0.5 + hsf) * 2.0
                b0f = jnp.where(u0 >= sf + dvf + 1.0, 1.0, 0.0)
                b1f = jnp.where(u1 >= jf + dhf + 1.0, 1.0, 0.0)
                onf = jnp.where((nv > 0) & (jv >= 0) & (jv < _NT), 1.0, 0.0)
                mf = onf * (1.0 - b0f) * (1.0 - b1f) * srow_f
                nl0 = u0 - (sf + dvf)
                nl1 = u1 - (jf + dhf)
                ob = pl.ds(rd * _NT + base, 16)
                cnt_b[ob] = mf.astype(jnp.int32)
                fo_b[ob] = fv * mf
                lo0_b[ob] = nl0 * mf
                lo1_b[ob] = nl1 * mf
                return c3

            def full_chunk(c, c3):
                base = c * 16
                jd = base + iota
                acc_n = zf
                acc_f = zf
                acc_0 = zf
                acc_1 = zf
                for r0 in (0, 1):
                    s = dest - dv - r0
                    rp = s - ws
                    sf = sf1 if r0 else sf0
                    srow_f = jnp.where(
                        (s >= 0) & (s < _NT) & (rp >= 0) & (rp < _SRC),
                        jnp.float32(1.0), jnp.float32(0.0))
                    thr0 = sf + dvf + 1.0
                    sub0 = sf + dvf + jnp.float32(r0)
                    for r1 in (0, 1):
                        off = rp * _NT + base - dh - r1 + _PAD
                        off = jnp.clip(off, 0, _PAD + _WLEN - 16)
                        jv = jd - dh - r1
                        nv = n_in[pl.ds(off, 16)]
                        fv = f_in[pl.ds(off, 16)]
                        l0v = l0_in[pl.ds(off, 16)]
                        l1v = l1_in[pl.ds(off, 16)]
                        jf = lax.convert_element_type(jv, jnp.float32)
                        u0 = ((sf + l0v) * 0.5 + vsf) * 2.0
                        u1 = ((jf + l1v) * 0.5 + hsf) * 2.0
                        b0f = jnp.where(u0 >= thr0, 1.0, 0.0)
                        b1f = jnp.where(u1 >= jf + dhf + 1.0, 1.0, 0.0)
                        m0f = b0f if r0 else 1.0 - b0f
                        m1f = b1f if r1 else 1.0 - b1f
                        onf = jnp.where((nv > 0) & (jv >= 0) & (jv < _NT),
                                        1.0, 0.0)
                        mf = onf * m0f * m1f * srow_f
                        nl0 = u0 - sub0
                        nl1 = u1 - (jf + dhf + jnp.float32(r1))
                        acc_n = acc_n + mf
                        acc_f = acc_f + fv * mf
                        acc_0 = acc_0 + nl0 * mf
                        acc_1 = acc_1 + nl1 * mf
                ob = pl.ds(rd * _NT + base, 16)
                cnt_b[ob] = jnp.minimum(acc_n, 1.0).astype(jnp.int32)
                fo_b[ob] = acc_f
                lo0_b[ob] = acc_0
                lo1_b[ob] = acc_1
                return c3

            def slow(_):
                lax.fori_loop(0, 64, full_chunk, 0)
                return 0

            def fast(_):
                lax.fori_loop(0, 64, base_chunk, 0)
                return 0

            lax.cond(anyflag, slow, fast, 0)
            return c2
        lax.fori_loop(0, _BAND, row_body, 0)

        out_off = pl.multiple_of(bs * _NT, 128)
        pltpu.sync_copy(cnt_b, no_hbm.at[pl.ds(out_off, _BLEN)])
        pltpu.sync_copy(fo_b, fo_hbm.at[pl.ds(out_off, _BLEN)])
        pltpu.sync_copy(lo0_b, lo0_hbm.at[pl.ds(out_off, _BLEN)])
        pltpu.sync_copy(lo1_b, lo1_hbm.at[pl.ds(out_off, _BLEN)])
        return carry
    lax.fori_loop(0, _NB, band_body, 0)


def kernel(images, psf_params, n_sources, locs, fluxes, vertical_shift,
           horizontal_shift):
    v = jnp.asarray(vertical_shift, jnp.int32)
    h = jnp.asarray(horizontal_shift, jnp.int32)
    sh = jnp.stack([v, h])

    img = pl.pallas_call(
        _img_body,
        out_shape=jax.ShapeDtypeStruct((_C, _H, _W), jnp.float32),
        in_specs=[pl.BlockSpec(memory_space=pltpu.SMEM),
                  pl.BlockSpec((_C, _H, _W), lambda: (0, 0, 0))],
        out_specs=pl.BlockSpec((_C, _H, _W), lambda: (0, 0, 0)),
    )(sh, images)

    sh16 = jnp.zeros((16,), jnp.int32).at[0].set(v).at[1].set(h)
    mesh = plsc.VectorSubcoreMesh(core_axis_name="c", subcore_axis_name="s")
    fvec = jax.ShapeDtypeStruct((_NT * _NT,), jnp.float32)
    sc_cat = functools.partial(
        pl.kernel,
        mesh=mesh,
        out_type=(jax.ShapeDtypeStruct((_NT * _NT,), jnp.int32),
                  fvec, fvec, fvec),
        scratch_types=[
            pltpu.VMEM((16,), jnp.int32),
            pltpu.VMEM((_WLEN + 2 * _PAD,), jnp.int32),
            pltpu.VMEM((_WLEN + 2 * _PAD,), jnp.float32),
            pltpu.VMEM((_WLEN + 2 * _PAD,), jnp.float32),
            pltpu.VMEM((_WLEN + 2 * _PAD,), jnp.float32),
            pltpu.VMEM((_BLEN,), jnp.int32),
            pltpu.VMEM((_BLEN,), jnp.float32),
            pltpu.VMEM((_BLEN,), jnp.float32),
            pltpu.VMEM((_BLEN,), jnp.float32),
        ],
    )(_sc_cat)
    n_out, lo0, lo1, f_out = sc_cat(
        sh16, n_sources.reshape(-1), locs[:, :, 0, 0].reshape(-1),
        locs[:, :, 0, 1].reshape(-1), fluxes.reshape(-1))

    locs_out = jnp.stack([lo0.reshape(_NT, _NT), lo1.reshape(_NT, _NT)],
                         axis=-1).reshape(_NT, _NT, 1, 2)
    return (img, psf_params, locs_out,
            f_out.reshape(_NT, _NT, 1, 1), n_out.reshape(_NT, _NT))


# speedup vs baseline: 1.1278x; 1.0096x over previous
import functools

import jax
import jax.numpy as jnp
from jax import lax
from jax.experimental import pallas as pl
from jax.experimental.pallas import tpu as pltpu
from jax.experimental.pallas import tpu_sc as plsc

_C, _H, _W = 5, 512, 512
_NT = 1024
_NW = 32
_RW = _NT // _NW
_BAND = 8
_NB = _RW // _BAND
_SRC = 9
_PAD = 16
_WLEN = _SRC * _NT
_BLEN = _BAND * _NT


def _img_body(sh_ref, img_ref, out_ref):
    v = sh_ref[0]
    h = sh_ref[1]
    x = img_ref[...]
    x = pltpu.roll(x, jnp.mod(v, _H), 1)
    x = pltpu.roll(x, jnp.mod(h, _W), 2)
    out_ref[...] = x


def _sc_cat(sh_hbm, n_hbm, l0_hbm, l1_hbm, f_hbm,
            no_hbm, lo0_hbm, lo1_hbm, fo_hbm,
            shv, n_in, l0_in, l1_in, f_in, cnt_b, lo0_b, lo1_b, fo_b):
    wid = lax.axis_index("s") * 2 + lax.axis_index("c")
    pltpu.sync_copy(sh_hbm, shv)
    shvec = shv[...]
    vs = shvec[0]
    hs = shvec[1]
    dv = 2 * vs
    dh = 2 * hs
    vsf = lax.convert_element_type(vs, jnp.float32)
    hsf = lax.convert_element_type(hs, jnp.float32)
    dvf = vsf * 2.0
    dhf = hsf * 2.0
    iota = lax.broadcasted_iota(jnp.int32, (16,), 0)
    zf = jnp.zeros((16,), jnp.float32)

    def band_body(b, carry):
        bs = wid * _RW + b * _BAND
        ws = jnp.clip(bs - dv - 1, 0, _NT - _SRC)
        src_off = pl.multiple_of(ws * _NT, 128)
        dst = pl.ds(_PAD, _WLEN)
        pltpu.sync_copy(n_hbm.at[pl.ds(src_off, _WLEN)], n_in.at[dst])
        pltpu.sync_copy(l0_hbm.at[pl.ds(src_off, _WLEN)], l0_in.at[dst])
        pltpu.sync_copy(l1_hbm.at[pl.ds(src_off, _WLEN)], l1_in.at[dst])
        pltpu.sync_copy(f_hbm.at[pl.ds(src_off, _WLEN)], f_in.at[dst])

        def row_body(rd, c2):
            dest = bs + rd
            s0 = dest - dv
            s1 = dest - dv - 1
            rp0 = s0 - ws
            rp1 = s1 - ws
            sf0 = lax.convert_element_type(s0, jnp.float32)
            sf1 = lax.convert_element_type(s1, jnp.float32)

            off1row = rp1 * _NT + _PAD
            srow_f = jnp.where(
                (s0 >= 0) & (s0 < _NT) & (rp0 >= 0) & (rp0 < _SRC),
                jnp.float32(1.0), jnp.float32(0.0))

            def base_chunk(c, acc):
                base = c * 16
                jd = base + iota
                off = jnp.clip(rp0 * _NT + base - dh + _PAD,
                               0, _PAD + _WLEN - 16)
                off1 = jnp.clip(off1row + base - dh, 0, _PAD + _WLEN - 16)
                jv = jd - dh
                nv = n_in[pl.ds(off, 16)]
                fv = f_in[pl.ds(off, 16)]
                l0v = l0_in[pl.ds(off, 16)]
                l1v = l1_in[pl.ds(off, 16)]
                l0s1 = l0_in[pl.ds(off1, 16)]
                jf = lax.convert_element_type(jv, jnp.float32)
                u0 = ((sf0 + l0v) * 0.5 + vsf) * 2.0
                u1 = ((jf + l1v) * 0.5 + hsf) * 2.0
                u0s1 = ((sf1 + l0s1) * 0.5 + vsf) * 2.0
                b0f = jnp.where(u0 >= sf0 + dvf + 1.0, 1.0, 0.0)
                b1f = jnp.where(u1 >= jf + dhf + 1.0, 1.0, 0.0)
                b0s1 = jnp.where(u0s1 >= sf1 + dvf + 1.0, 1.0, 0.0)
                colf = jnp.where((jv >= 0) & (jv < _NT), 1.0, 0.0)
                onf = jnp.where(nv > 0, 1.0, 0.0) * colf
                mf = onf * (1.0 - b0f) * (1.0 - b1f) * srow_f
                nl0 = u0 - (sf0 + dvf)
                nl1 = u1 - (jf + dhf)
                ob = pl.ds(rd * _NT + base, 16)
                cnt_b[ob] = mf.astype(jnp.int32)
                fo_b[ob] = fv * mf
                lo0_b[ob] = nl0 * mf
                lo1_b[ob] = nl1 * mf
                return jnp.maximum(acc, jnp.maximum(b1f * onf, b0s1 * colf))

            accfl = lax.fori_loop(0, 64, base_chunk, zf)
            accs = accfl[0]
            for _k in range(1, 16):
                accs = accs + accfl[_k]
            anyflag = accs > 0.0

            def full_chunk(c, c3):
                base = c * 16
                jd = base + iota
                acc_n = zf
                acc_f = zf
                acc_0 = zf
                acc_1 = zf
                for r0 in (0, 1):
                    s = dest - dv - r0
                    rp = s - ws
                    sf = sf1 if r0 else sf0
                    srow_f = jnp.where(
                        (s >= 0) & (s < _NT) & (rp >= 0) & (rp < _SRC),
                        jnp.float32(1.0), jnp.float32(0.0))
                    thr0 = sf + dvf + 1.0
                    sub0 = sf + dvf + jnp.float32(r0)
                    for r1 in (0, 1):
                        off = rp * _NT + base - dh - r1 + _PAD
                        off = jnp.clip(off, 0, _PAD + _WLEN - 16)
                        jv = jd - dh - r1
                        nv = n_in[pl.ds(off, 16)]
                        fv = f_in[pl.ds(off, 16)]
                        l0v = l0_in[pl.ds(off, 16)]
                        l1v = l1_in[pl.ds(off, 16)]
                        jf = lax.convert_element_type(jv, jnp.float32)
                        u0 = ((sf + l0v) * 0.5 + vsf) * 2.0
                        u1 = ((jf + l1v) * 0.5 + hsf) * 2.0
                        b0f = jnp.where(u0 >= thr0, 1.0, 0.0)
                        b1f = jnp.where(u1 >= jf + dhf + 1.0, 1.0, 0.0)
                        m0f = b0f if r0 else 1.0 - b0f
                        m1f = b1f if r1 else 1.0 - b1f
                        onf = jnp.where((nv > 0) & (jv >= 0) & (jv < _NT),
                                        1.0, 0.0)
                        mf = onf * m0f * m1f * srow_f
                        nl0 = u0 - sub0
                        nl1 = u1 - (jf + dhf + jnp.float32(r1))
                        acc_n = acc_n + mf
                        acc_f = acc_f + fv * mf
                        acc_0 = acc_0 + nl0 * mf
                        acc_1 = acc_1 + nl1 * mf
                ob = pl.ds(rd * _NT + base, 16)
                cnt_b[ob] = jnp.minimum(acc_n, 1.0).astype(jnp.int32)
                fo_b[ob] = acc_f
                lo0_b[ob] = acc_0
                lo1_b[ob] = acc_1
                return c3

            def slow(_):
                lax.fori_loop(0, 64, full_chunk, 0)
                return 0

            def noop(_):
                return 0

            lax.cond(anyflag, slow, noop, 0)
            return c2
        lax.fori_loop(0, _BAND, row_body, 0)

        out_off = pl.multiple_of(bs * _NT, 128)
        pltpu.sync_copy(cnt_b, no_hbm.at[pl.ds(out_off, _BLEN)])
        pltpu.sync_copy(fo_b, fo_hbm.at[pl.ds(out_off, _BLEN)])
        pltpu.sync_copy(lo0_b, lo0_hbm.at[pl.ds(out_off, _BLEN)])
        pltpu.sync_copy(lo1_b, lo1_hbm.at[pl.ds(out_off, _BLEN)])
        return carry
    lax.fori_loop(0, _NB, band_body, 0)


def kernel(images, psf_params, n_sources, locs, fluxes, vertical_shift,
           horizontal_shift):
    v = jnp.asarray(vertical_shift, jnp.int32)
    h = jnp.asarray(horizontal_shift, jnp.int32)
    sh = jnp.stack([v, h])

    img = pl.pallas_call(
        _img_body,
        out_shape=jax.ShapeDtypeStruct((_C, _H, _W), jnp.float32),
        in_specs=[pl.BlockSpec(memory_space=pltpu.SMEM),
                  pl.BlockSpec((_C, _H, _W), lambda: (0, 0, 0))],
        out_specs=pl.BlockSpec((_C, _H, _W), lambda: (0, 0, 0)),
    )(sh, images)

    sh16 = jnp.zeros((16,), jnp.int32).at[0].set(v).at[1].set(h)
    mesh = plsc.VectorSubcoreMesh(core_axis_name="c", subcore_axis_name="s")
    fvec = jax.ShapeDtypeStruct((_NT * _NT,), jnp.float32)
    sc_cat = functools.partial(
        pl.kernel,
        mesh=mesh,
        out_type=(jax.ShapeDtypeStruct((_NT * _NT,), jnp.int32),
                  fvec, fvec, fvec),
        scratch_types=[
            pltpu.VMEM((16,), jnp.int32),
            pltpu.VMEM((_WLEN + 2 * _PAD,), jnp.int32),
            pltpu.VMEM((_WLEN + 2 * _PAD,), jnp.float32),
            pltpu.VMEM((_WLEN + 2 * _PAD,), jnp.float32),
            pltpu.VMEM((_WLEN + 2 * _PAD,), jnp.float32),
            pltpu.VMEM((_BLEN,), jnp.int32),
            pltpu.VMEM((_BLEN,), jnp.float32),
            pltpu.VMEM((_BLEN,), jnp.float32),
            pltpu.VMEM((_BLEN,), jnp.float32),
        ],
    )(_sc_cat)
    n_out, lo0, lo1, f_out = sc_cat(
        sh16, n_sources.reshape(-1), locs[:, :, 0, 0].reshape(-1),
        locs[:, :, 0, 1].reshape(-1), fluxes.reshape(-1))

    locs_out = jnp.stack([lo0.reshape(_NT, _NT), lo1.reshape(_NT, _NT)],
                         axis=-1).reshape(_NT, _NT, 1, 2)
    return (img, psf_params, locs_out,
            f_out.reshape(_NT, _NT, 1, 1), n_out.reshape(_NT, _NT))
